# baseline (speedup 1.0000x reference)
import functools

import jax
import jax.numpy as jnp
from jax import lax
from jax.experimental import pallas as pl
from jax.experimental.pallas import tpu as pltpu
from jax.experimental.pallas import tpu_sc as plsc


def _pattern_body(vx_ref, g_ref, vy_ref, out_ref):
    H = vx_ref.shape[0]
    W = vy_ref.shape[0]
    vx = vx_ref[...]
    gx = g_ref[:, 0:1]
    gy = g_ref[:, 1:2]
    wx = gx * vx * jax.lax.rsqrt(jnp.sum(vx * vx, axis=1, keepdims=True))
    vy = vy_ref[...]
    wy = gy * vy * jax.lax.rsqrt(jnp.sum(vy * vy, axis=1, keepdims=True))
    xblock = jnp.tile(wx, (W, 1))
    yblock = jnp.repeat(wy, H, axis=0)
    out_ref[...] = jnp.concatenate([xblock, yblock], axis=1)


def _bcast_body(p_ref, out_ref):
    out_ref[0] = p_ref[...]


def kernel(inp, vx, gx, vy, gy):
    b = inp.shape[0]
    H, D = vx.shape
    W = vy.shape[0]
    rows, width = W * H, 2 * D
    g2 = jnp.concatenate([gx, gy], axis=1)

    full = lambda s: pl.BlockSpec(s, lambda: (0,) * len(s))
    pattern = pl.pallas_call(
        _pattern_body,
        in_specs=[full((H, D)), full((H, 2)), full((W, D))],
        out_specs=full((rows, width)),
        out_shape=jax.ShapeDtypeStruct((rows, width), jnp.float32),
    )(vx, g2, vy)

    b_sc = (26 * b) // 64
    b_tc = b - b_sc

    info = plsc.get_sparse_core_info()
    NW = info.num_cores * info.num_subcores
    rpw = rows // NW

    @functools.partial(
        pl.kernel,
        mesh=plsc.VectorSubcoreMesh(core_axis_name="c", subcore_axis_name="s"),
        out_type=jax.ShapeDtypeStruct((b_sc, rows, width), jnp.float32),
        scratch_types=[
            pltpu.VMEM((rpw, width), jnp.float32),
            pltpu.SemaphoreType.DMA,
        ],
    )
    def sc_broadcast(p_hbm, out_hbm, chunk, sem):
        wid = lax.axis_index("s") * info.num_cores + lax.axis_index("c")
        rbase = wid * rpw
        pltpu.sync_copy(p_hbm.at[pl.ds(rbase, rpw)], chunk)
        descs = [
            pltpu.async_copy(chunk, out_hbm.at[i, pl.ds(rbase, rpw)], sem)
            for i in range(b_sc)
        ]
        for d in descs:
            d.wait()

    sc_out = sc_broadcast(pattern)

    tc_out = pl.pallas_call(
        _bcast_body,
        grid=(b_tc,),
        in_specs=[pl.BlockSpec((rows, width), lambda i: (0, 0))],
        out_specs=pl.BlockSpec((1, rows, width), lambda i: (i, 0, 0)),
        out_shape=jax.ShapeDtypeStruct((b_tc, rows, width), jnp.float32),
    )(pattern)

    return jnp.concatenate([sc_out, tc_out], axis=0)


# speedup vs baseline: 2.3442x; 2.3442x over previous
import functools

import jax
import jax.numpy as jnp
from jax import lax
from jax.experimental import pallas as pl
from jax.experimental.pallas import tpu as pltpu
from jax.experimental.pallas import tpu_sc as plsc


def _pattern_body(vx_ref, g_ref, vy_ref, out_ref):
    H = vx_ref.shape[0]
    W = vy_ref.shape[0]
    vx = vx_ref[...]
    gx = g_ref[:, 0:1]
    gy = g_ref[:, 1:2]
    wx = gx * vx * jax.lax.rsqrt(jnp.sum(vx * vx, axis=1, keepdims=True))
    vy = vy_ref[...]
    wy = gy * vy * jax.lax.rsqrt(jnp.sum(vy * vy, axis=1, keepdims=True))
    xblock = jnp.tile(wx, (W, 1))
    yblock = jnp.repeat(wy, H, axis=0)
    out_ref[...] = jnp.concatenate([xblock, yblock], axis=1)


def kernel(inp, vx, gx, vy, gy):
    b = inp.shape[0]
    H, D = vx.shape
    W = vy.shape[0]
    rows, width = W * H, 2 * D
    TR, TC = rows // 8, width // 128
    g2 = jnp.concatenate([gx, gy], axis=1)

    full = lambda s: pl.BlockSpec(s, lambda: (0,) * len(s))
    pattern = pl.pallas_call(
        _pattern_body,
        in_specs=[full((H, D)), full((H, 2)), full((W, D))],
        out_specs=full((rows, width)),
        out_shape=jax.ShapeDtypeStruct((rows, width), jnp.float32),
    )(vx, g2, vy)
    p4 = pattern.reshape(TR, 8, TC, 128).transpose(0, 2, 1, 3)

    info = plsc.get_sparse_core_info()
    NW = info.num_cores * info.num_subcores
    rpw = rows // NW
    tpw = rpw // 8

    @functools.partial(
        pl.kernel,
        mesh=plsc.VectorSubcoreMesh(core_axis_name="c", subcore_axis_name="s"),
        out_type=jax.ShapeDtypeStruct((b, TR, TC, 8, 128), jnp.float32),
        scratch_types=[
            pltpu.VMEM((tpw, TC, 8, 128), jnp.float32),
            pltpu.SemaphoreType.DMA,
        ],
    )
    def sc_broadcast(p4_hbm, out_hbm, chunk, sem):
        wid = lax.axis_index("s") * info.num_cores + lax.axis_index("c")
        tbase = wid * tpw
        pltpu.sync_copy(p4_hbm.at[pl.ds(tbase, tpw)], chunk)
        descs = [
            pltpu.async_copy(chunk, out_hbm.at[i, pl.ds(tbase, tpw)], sem)
            for i in range(b)
        ]
        for d in descs:
            d.wait()

    x5 = sc_broadcast(p4)
    return x5.transpose(0, 1, 3, 2, 4).reshape(b, rows, width)
